# Initial kernel scaffold; baseline (speedup 1.0000x reference)
import jax
import jax.numpy as jnp
from jax.experimental import pallas as pl


def kernel(word_id, positive_context_ids, negative_context_ids, W_word, W_ctx):
    raise NotImplementedError("write your pallas kernel here")


# speedup vs baseline: 1.4982x; 1.4982x over previous
import jax
import jax.numpy as jnp
from jax import lax
from jax.experimental import pallas as pl
from jax.experimental.pallas import tpu as pltpu, tpu_sc as plsc

B = 16384
V = 100000
D = 128
P = 20
N = 50
C = P + N

NC = 2
NS = 16
NW = NC * NS
BPW = B // NW
SR = 64
NSC = BPW // SR
E = SR * C
L = 16


def _w2v_body(cids_hbm, wid_hbm, wtab_hbm, ctab_hbm, out_hbm,
              cids_v, widx_v, wrows_v, crows_v, dots_v, sem):
    wid = lax.axis_index("s") * NC + lax.axis_index("c")
    base = wid * BPW

    def superchunk(sc, _):
        row0 = base + sc * SR
        pltpu.sync_copy(wid_hbm.at[pl.ds(row0, SR)], widx_v)
        pltpu.sync_copy(cids_hbm.at[pl.ds(row0, SR), :], cids_v)
        pltpu.async_copy(wtab_hbm.at[widx_v], wrows_v, sem).wait()

        NG = (C + L - 1) // L
        lane = lax.iota(jnp.int32, L)
        rowidx = [lane + g * L for g in range(NG)]

        def row_body(r, _):
            pltpu.async_copy(
                ctab_hbm.at[cids_v.at[r]], crows_v.at[pl.ds(0, C), :], sem
            ).wait()
            accs = [jnp.zeros((L,), jnp.float32) for _ in range(NG)]
            for k in range(D // L):
                wvk = wrows_v[r, pl.ds(k * L, L)]
                for dd in range(L):
                    d = k * L + dd
                    w_d = wvk[dd]
                    col = jnp.full((L,), d, jnp.int32)
                    for g in range(NG):
                        cv = plsc.load_gather(crows_v, [rowidx[g], col])
                        accs[g] = accs[g] + cv * w_d
            for g in range(NG):
                sig = 1.0 / (1.0 + jnp.exp(-accs[g]))
                dots_v[pl.ds(r * C + g * L, L)] = sig
            return 0

        lax.fori_loop(0, SR, row_body, 0)
        pltpu.sync_copy(dots_v.at[pl.ds(0, E)], out_hbm.at[pl.ds(row0 * C, E)])
        return 0

    lax.fori_loop(0, NSC, superchunk, 0)


def kernel(word_id, positive_context_ids, negative_context_ids, W_word, W_ctx):
    ctx_ids = jnp.concatenate(
        [positive_context_ids, negative_context_ids], axis=1
    ).astype(jnp.int32)
    wid32 = word_id.astype(jnp.int32)

    mesh = plsc.VectorSubcoreMesh(core_axis_name="c", subcore_axis_name="s")
    run = pl.kernel(
        _w2v_body,
        out_type=jax.ShapeDtypeStruct((B * C,), jnp.float32),
        mesh=mesh,
        compiler_params=pltpu.CompilerParams(needs_layout_passes=False),
        scratch_types=[
            pltpu.VMEM((SR, C), jnp.int32),
            pltpu.VMEM((SR,), jnp.int32),
            pltpu.VMEM((SR, D), jnp.float32),
            pltpu.VMEM(((C + L - 1) // L * L, D), jnp.float32),
            pltpu.VMEM((E + L,), jnp.float32),
            pltpu.SemaphoreType.DMA,
        ],
    )
    out = run(ctx_ids, wid32, W_word, W_ctx).reshape(B, C)
    return out[:, :P], out[:, P:]


# speedup vs baseline: 1.6564x; 1.1056x over previous
import jax
import jax.numpy as jnp
from jax import lax
from jax.experimental import pallas as pl
from jax.experimental.pallas import tpu as pltpu, tpu_sc as plsc

B = 16384
V = 100000
D = 128
P = 20
N = 50
C = P + N

NC = 2
NS = 16
NW = NC * NS
BPW = B // NW
SR = 64
NSC = BPW // SR
E = SR * C
L = 16


NB = 2
CP = (C + L - 1) // L * L
NG = CP // L


def _w2v_body(cids_hbm, wid_hbm, wtab_hbm, ctab_hbm, out_hbm,
              cids_v, widx_v, wrows_v, crows_v, dots_v, wsem, *sems):
    wid = lax.axis_index("s") * NC + lax.axis_index("c")
    base = wid * BPW

    def gather_row(r, b):
        return pltpu.make_async_copy(
            ctab_hbm.at[cids_v.at[r]], crows_v.at[b, pl.ds(0, C), :], sems[b]
        )

    def superchunk(sc, _):
        row0 = base + sc * SR
        pltpu.sync_copy(wid_hbm.at[pl.ds(row0, SR)], widx_v)
        pltpu.sync_copy(cids_hbm.at[pl.ds(row0, SR), :], cids_v)
        wcopy = pltpu.make_async_copy(wtab_hbm.at[widx_v], wrows_v, wsem)
        wcopy.start()
        for b in range(NB):
            gather_row(b, b).start()
        wcopy.wait()

        lane = lax.iota(jnp.int32, L)
        rowidx = [lane + g * L for g in range(NG)]

        def step(t, _):
            for b in range(NB):
                r = t * NB + b
                gather_row(r, b).wait()
                accs = [jnp.zeros((L,), jnp.float32) for _ in range(NG)]
                for k in range(D // L):
                    wvk = wrows_v[r, pl.ds(k * L, L)]
                    for dd in range(L):
                        d = k * L + dd
                        w_d = wvk[dd]
                        col = jnp.full((L,), d, jnp.int32)
                        for g in range(NG):
                            cv = plsc.load_gather(crows_v.at[b], [rowidx[g], col])
                            accs[g] = accs[g] + cv * w_d
                for g in range(NG):
                    sig = 1.0 / (1.0 + jnp.exp(-accs[g]))
                    dots_v[pl.ds(r * C + g * L, L)] = sig

                @pl.when(r + NB < SR)
                def _():
                    gather_row(r + NB, b).start()

            return 0

        lax.fori_loop(0, SR // NB, step, 0)
        pltpu.sync_copy(dots_v.at[pl.ds(0, E)], out_hbm.at[pl.ds(row0 * C, E)])
        return 0

    lax.fori_loop(0, NSC, superchunk, 0)


def kernel(word_id, positive_context_ids, negative_context_ids, W_word, W_ctx):
    ctx_ids = jnp.concatenate(
        [positive_context_ids, negative_context_ids], axis=1
    ).astype(jnp.int32)
    wid32 = word_id.astype(jnp.int32)

    mesh = plsc.VectorSubcoreMesh(core_axis_name="c", subcore_axis_name="s")
    run = pl.kernel(
        _w2v_body,
        out_type=jax.ShapeDtypeStruct((B * C,), jnp.float32),
        mesh=mesh,
        compiler_params=pltpu.CompilerParams(needs_layout_passes=False),
        scratch_types=[
            pltpu.VMEM((SR, C), jnp.int32),
            pltpu.VMEM((SR,), jnp.int32),
            pltpu.VMEM((SR, D), jnp.float32),
            pltpu.VMEM((NB, CP, D), jnp.float32),
            pltpu.VMEM((E + L,), jnp.float32),
            pltpu.SemaphoreType.DMA,
        ] + [pltpu.SemaphoreType.DMA] * NB,
    )
    out = run(ctx_ids, wid32, W_word, W_ctx).reshape(B, C)
    return out[:, :P], out[:, P:]


# speedup vs baseline: 3.1702x; 1.9139x over previous
import jax
import jax.numpy as jnp
from jax import lax
from jax.experimental import pallas as pl
from jax.experimental.pallas import tpu as pltpu, tpu_sc as plsc

B = 16384
V = 100000
D = 128
P = 20
N = 50
C = P + N

NC = 2
NS = 16
NW = NC * NS
BPW = B // NW
SR = 64
NSC = BPW // SR
E = SR * C
L = 16


NB = 2
CP = (C + L - 1) // L * L
NG = CP // L


def _w2v_body(cids_hbm, wid_hbm, wtab_hbm, ctab_hbm, out_hbm,
              cids_v, widx_v, wrows_v, crows_v, dots_v, wsem, *sems):
    wid = lax.axis_index("s") * NC + lax.axis_index("c")
    base = wid * BPW

    def gather_row(r, b):
        return pltpu.make_async_copy(
            ctab_hbm.at[cids_v.at[r]], crows_v.at[b, pl.ds(0, C), :], sems[b]
        )

    def superchunk(sc, _):
        row0 = base + sc * SR
        pltpu.sync_copy(wid_hbm.at[pl.ds(row0, SR)], widx_v)
        pltpu.sync_copy(cids_hbm.at[pl.ds(row0, SR), :], cids_v)
        wcopy = pltpu.make_async_copy(wtab_hbm.at[widx_v], wrows_v, wsem)
        wcopy.start()
        for b in range(NB):
            gather_row(b, b).start()
        wcopy.wait()

        lane = lax.iota(jnp.int32, L)
        rowidx = [lane + g * L for g in range(NG)]

        def step(t, _):
            for b in range(NB):
                r = t * NB + b
                gather_row(r, b).wait()
                accs = [jnp.zeros((L,), jnp.float32) for _ in range(NG)]
                rv = jnp.full((L,), r, jnp.int32)
                for s in range(D):
                    dv = lane + s
                    dv = jnp.where(dv >= D, dv - D, dv)
                    wv = plsc.load_gather(wrows_v, [rv, dv])
                    for g in range(NG):
                        cv = plsc.load_gather(crows_v.at[b], [rowidx[g], dv])
                        accs[g] = accs[g] + cv * wv
                for g in range(NG):
                    sig = 1.0 / (1.0 + jnp.exp(-accs[g]))
                    dots_v[pl.ds(r * C + g * L, L)] = sig

                @pl.when(r + NB < SR)
                def _():
                    gather_row(r + NB, b).start()

            return 0

        lax.fori_loop(0, SR // NB, step, 0)
        pltpu.sync_copy(dots_v.at[pl.ds(0, E)], out_hbm.at[pl.ds(row0 * C, E)])
        return 0

    lax.fori_loop(0, NSC, superchunk, 0)


def kernel(word_id, positive_context_ids, negative_context_ids, W_word, W_ctx):
    ctx_ids = jnp.concatenate(
        [positive_context_ids, negative_context_ids], axis=1
    ).astype(jnp.int32)
    wid32 = word_id.astype(jnp.int32)

    mesh = plsc.VectorSubcoreMesh(core_axis_name="c", subcore_axis_name="s")
    run = pl.kernel(
        _w2v_body,
        out_type=jax.ShapeDtypeStruct((B * C,), jnp.float32),
        mesh=mesh,
        compiler_params=pltpu.CompilerParams(needs_layout_passes=False),
        scratch_types=[
            pltpu.VMEM((SR, C), jnp.int32),
            pltpu.VMEM((SR,), jnp.int32),
            pltpu.VMEM((SR, D), jnp.float32),
            pltpu.VMEM((NB, CP, D), jnp.float32),
            pltpu.VMEM((E + L,), jnp.float32),
            pltpu.SemaphoreType.DMA,
        ] + [pltpu.SemaphoreType.DMA] * NB,
    )
    out = run(ctx_ids, wid32, W_word, W_ctx).reshape(B, C)
    return out[:, :P], out[:, P:]


# speedup vs baseline: 5.9191x; 1.8671x over previous
import jax
import jax.numpy as jnp
from jax import lax
from jax.experimental import pallas as pl
from jax.experimental.pallas import tpu as pltpu, tpu_sc as plsc

B = 16384
V = 100000
D = 128
P = 20
N = 50
C = P + N

NC = 2
NS = 16
NW = NC * NS
BPW = B // NW
SR = 64
NSC = BPW // SR
E = SR * C
L = 16


NB = 2
CP = (C + L - 1) // L * L
NG = CP // L


def _w2v_body(cids_hbm, wid_hbm, wtab_hbm, ctab_hbm, out_hbm,
              cids_v, widx_v, wrows_v, crows_v, dots_v, mat_v, wsem, *sems):
    wid = lax.axis_index("s") * NC + lax.axis_index("c")
    base = wid * BPW

    def gather_row(r, b):
        return pltpu.make_async_copy(
            ctab_hbm.at[cids_v.at[r]], crows_v.at[b, pl.ds(0, C), :], sems[b]
        )

    def superchunk(sc, _):
        row0 = base + sc * SR
        pltpu.sync_copy(wid_hbm.at[pl.ds(row0, SR)], widx_v)
        pltpu.sync_copy(cids_hbm.at[pl.ds(row0, SR), :], cids_v)
        wcopy = pltpu.make_async_copy(wtab_hbm.at[widx_v], wrows_v, wsem)
        wcopy.start()
        for b in range(NB):
            gather_row(b, b).start()
        wcopy.wait()

        lane = lax.iota(jnp.int32, L)
        rot = []
        for i in range(L):
            dv = lane + i
            rot.append(jnp.where(dv >= L, dv - L, dv))

        def step(t, _):
            for b in range(NB):
                r = t * NB + b
                gather_row(r, b).wait()
                wv = [wrows_v[r, pl.ds(k * L, L)] for k in range(D // L)]
                for g in range(NG):
                    for i in range(L):
                        j = g * L + i
                        acc = crows_v[b, j, pl.ds(0, L)] * wv[0]
                        for k in range(1, D // L):
                            acc = acc + crows_v[b, j, pl.ds(k * L, L)] * wv[k]
                        mat_v[i, :] = acc
                    tot = plsc.load_gather(mat_v, [lane, rot[0]])
                    for i in range(1, L):
                        tot = tot + plsc.load_gather(mat_v, [lane, rot[i]])
                    sig = 1.0 / (1.0 + jnp.exp(-tot))
                    dots_v[pl.ds(r * C + g * L, L)] = sig

                @pl.when(r + NB < SR)
                def _():
                    gather_row(r + NB, b).start()

            return 0

        lax.fori_loop(0, SR // NB, step, 0)
        pltpu.sync_copy(dots_v.at[pl.ds(0, E)], out_hbm.at[pl.ds(row0 * C, E)])
        return 0

    lax.fori_loop(0, NSC, superchunk, 0)


def kernel(word_id, positive_context_ids, negative_context_ids, W_word, W_ctx):
    ctx_ids = jnp.concatenate(
        [positive_context_ids, negative_context_ids], axis=1
    ).astype(jnp.int32)
    wid32 = word_id.astype(jnp.int32)

    mesh = plsc.VectorSubcoreMesh(core_axis_name="c", subcore_axis_name="s")
    run = pl.kernel(
        _w2v_body,
        out_type=jax.ShapeDtypeStruct((B * C,), jnp.float32),
        mesh=mesh,
        compiler_params=pltpu.CompilerParams(needs_layout_passes=False),
        scratch_types=[
            pltpu.VMEM((SR, C), jnp.int32),
            pltpu.VMEM((SR,), jnp.int32),
            pltpu.VMEM((SR, D), jnp.float32),
            pltpu.VMEM((NB, CP, D), jnp.float32),
            pltpu.VMEM((E + L,), jnp.float32),
            pltpu.VMEM((L, L), jnp.float32),
            pltpu.SemaphoreType.DMA,
        ] + [pltpu.SemaphoreType.DMA] * NB,
    )
    out = run(ctx_ids, wid32, W_word, W_ctx).reshape(B, C)
    return out[:, :P], out[:, P:]
